# Initial kernel scaffold; baseline (speedup 1.0000x reference)
import jax
import jax.numpy as jnp
from jax.experimental import pallas as pl
from jax.experimental.pallas import tpu as pltpu

_V = 1000000
_B = 32
_SEED = 42
_PRIOR = 4
_KHASH = 4
_TOPP = 0.9
_ND = 8
_TINY = jnp.float32(1.1754943508222875e-38)

_ROWS = 8
_COLS = _V // _ROWS

_OUT_BS = 65536


def _rotl(x, r):
    return (x << jnp.uint32(r)) | (x >> jnp.uint32(32 - r))


def _threefry_mix(k1, k2, n):
    ks2 = k1 ^ k2 ^ jnp.uint32(0x1BD11BDA)
    x0 = jnp.zeros_like(n) + k1
    x1 = n + k2
    for r in (13, 15, 26, 6):
        x0 = x0 + x1
        x1 = _rotl(x1, r) ^ x0
    x0 = x0 + k2
    x1 = x1 + ks2 + jnp.uint32(1)
    for r in (17, 29, 16, 24):
        x0 = x0 + x1
        x1 = _rotl(x1, r) ^ x0
    x0 = x0 + ks2
    x1 = x1 + k1 + jnp.uint32(2)
    for r in (13, 15, 26, 6):
        x0 = x0 + x1
        x1 = _rotl(x1, r) ^ x0
    x0 = x0 + k1
    x1 = x1 + k2 + jnp.uint32(3)
    for r in (17, 29, 16, 24):
        x0 = x0 + x1
        x1 = _rotl(x1, r) ^ x0
    x0 = x0 + k2
    x1 = x1 + ks2 + jnp.uint32(4)
    for r in (13, 15, 26, 6):
        x0 = x0 + x1
        x1 = _rotl(x1, r) ^ x0
    x0 = x0 + ks2
    x1 = x1 + k1 + jnp.uint32(5)
    return x0 ^ x1


def _mix32(x):
    x = x ^ (x >> jnp.uint32(16))
    x = x * jnp.uint32(0x7FEB352D)
    x = x ^ (x >> jnp.uint32(15))
    x = x * jnp.uint32(0x846CA68B)
    x = x ^ (x >> jnp.uint32(16))
    return x


def _hcomb(h, v):
    return _mix32(h ^ (_mix32(v) + jnp.uint32(0x9E3779B9) + (h << jnp.uint32(6)) + (h >> jnp.uint32(2))))


def _draws_kernel(cutoff_ref, key_ref, sp_ref, out_ref):
    b = pl.program_id(0)
    cutoff = cutoff_ref[b]
    k1 = key_ref[0]
    k2 = key_ref[1]
    sp = sp_ref[0]
    pos = (jax.lax.broadcasted_iota(jnp.int32, (_ROWS, _COLS), 0) * _COLS
           + jax.lax.broadcasted_iota(jnp.int32, (_ROWS, _COLS), 1))
    keep = pos <= cutoff
    for d in range(_ND):
        base = (jnp.uint32(d * _B) + b.astype(jnp.uint32)) * jnp.uint32(_V)
        n = base + pos.astype(jnp.uint32)
        bits = _threefry_mix(k1, k2, n)
        fb = (bits >> jnp.uint32(9)) | jnp.uint32(0x3F800000)
        f = jax.lax.bitcast_convert_type(fb, jnp.float32) - jnp.float32(1.0)
        u = jnp.maximum(_TINY, f + _TINY)
        val = sp / (-jnp.log(u))
        val = jnp.where(keep, val, jnp.float32(-1.0))
        m = jnp.max(val)
        idx = jnp.min(jnp.where(val == m, pos, jnp.int32(_V)))
        out_ref[0, d, :] = jnp.broadcast_to(idx.reshape(1), (128,))


def _emit_kernel(cand_ref, tail_ref, ridx_ref, out_ref):
    cand = cand_ref[...].astype(jnp.uint32)
    lane = jax.lax.broadcasted_iota(jnp.int32, (_B, 128), 1)
    seed = jnp.full((_B, 128), _SEED, dtype=jnp.uint32)
    h = _mix32(seed ^ _mix32(ridx_ref[...].astype(jnp.uint32)))
    for t in range(_PRIOR):
        v = jnp.broadcast_to(tail_ref[:, t:t + 1].astype(jnp.uint32), (_B, 128))
        h = _hcomb(h, v)
    base = _hcomb(_mix32(seed), h)
    cs = _hcomb(base, cand)
    u01 = jnp.clip((cs >> jnp.uint32(8)).astype(jnp.float32) * jnp.float32(1.0 / 16777216.0),
                   jnp.float32(1e-7), jnp.float32(1.0 - 1e-7))
    u01 = jnp.where(lane < _ND, u01, jnp.float32(-1.0))
    m = jnp.max(u01, axis=1, keepdims=True)
    first = jnp.min(jnp.where(u01 == m, lane, jnp.int32(128)), axis=1, keepdims=True)
    tok = jnp.sum(jnp.where(lane == first, cand_ref[...], 0), axis=1, keepdims=True)
    j = pl.program_id(0)
    gcol = jax.lax.broadcasted_iota(jnp.int32, (_B, _OUT_BS), 1) + j * _OUT_BS
    out_ref[...] = jnp.where(gcol == tok, jnp.float32(100000.0), jnp.float32(1e-05))


def kernel(input_ids, logits):
    B, V = logits.shape
    probs = jax.nn.softmax(logits, axis=-1)
    iota = jax.lax.broadcasted_iota(jnp.int32, (B, V), 1)
    sneg, order = jax.lax.sort((-probs, iota), dimension=-1, is_stable=True, num_keys=1)
    sp = -sneg
    cum = jnp.cumsum(sp, axis=-1)
    cutoff = jnp.minimum(jnp.sum((cum < _TOPP).astype(jnp.int32), axis=-1), V - 1)

    skey = jax.random.key(1)
    kd, kr = jax.random.split(skey)
    keydata = jax.random.key_data(kd).astype(jnp.uint32)

    pstar = pl.pallas_call(
        _draws_kernel,
        grid_spec=pltpu.PrefetchScalarGridSpec(
            num_scalar_prefetch=2,
            grid=(B,),
            in_specs=[pl.BlockSpec((1, _ROWS, _COLS), lambda b, *_: (b, 0, 0))],
            out_specs=pl.BlockSpec((1, _ND, 128), lambda b, *_: (b, 0, 0)),
        ),
        out_shape=jax.ShapeDtypeStruct((B, _ND, 128), jnp.int32),
    )(cutoff.astype(jnp.int32), keydata, sp.reshape(B, _ROWS, _COLS))
    draws = pstar[:, :, 0]

    cand = jnp.take_along_axis(order, draws, axis=-1)
    r_idx = jax.random.randint(kr, (B,), 0, _KHASH, dtype=jnp.int32)

    cand_p = jnp.pad(cand, ((0, 0), (0, 128 - _ND)))
    tail_p = jnp.pad(input_ids[:, -_PRIOR:], ((0, 0), (0, 128 - _PRIOR)))
    ridx_p = jnp.broadcast_to(r_idx[:, None], (B, 128))

    nblk = (V + _OUT_BS - 1) // _OUT_BS
    out = pl.pallas_call(
        _emit_kernel,
        grid=(nblk,),
        in_specs=[
            pl.BlockSpec((B, 128), lambda j: (0, 0)),
            pl.BlockSpec((B, 128), lambda j: (0, 0)),
            pl.BlockSpec((B, 128), lambda j: (0, 0)),
        ],
        out_specs=pl.BlockSpec((B, _OUT_BS), lambda j: (0, j)),
        out_shape=jax.ShapeDtypeStruct((B, V), jnp.float32),
    )(cand_p, tail_p, ridx_p)
    return out


# speedup vs baseline: 1.0750x; 1.0750x over previous
import jax
import jax.numpy as jnp
import numpy as np
from jax.experimental import pallas as pl
from jax.experimental.pallas import tpu as pltpu

_V = 1000000
_B = 32
_SEED = 42
_PRIOR = 4
_KHASH = 4
_TOPP = 0.9
_ND = 8
_TINY = np.float32(1.1754943508222875e-38)

_ROWS = 8
_COLS = _V // _ROWS

_OUT_BS = 65536


def _rotl(x, r):
    return (x << jnp.uint32(r)) | (x >> jnp.uint32(32 - r))


def _threefry_mix(k1, k2, n):
    ks2 = k1 ^ k2 ^ jnp.uint32(0x1BD11BDA)
    x0 = jnp.zeros_like(n) + k1
    x1 = n + k2
    for r in (13, 15, 26, 6):
        x0 = x0 + x1
        x1 = _rotl(x1, r) ^ x0
    x0 = x0 + k2
    x1 = x1 + ks2 + jnp.uint32(1)
    for r in (17, 29, 16, 24):
        x0 = x0 + x1
        x1 = _rotl(x1, r) ^ x0
    x0 = x0 + ks2
    x1 = x1 + k1 + jnp.uint32(2)
    for r in (13, 15, 26, 6):
        x0 = x0 + x1
        x1 = _rotl(x1, r) ^ x0
    x0 = x0 + k1
    x1 = x1 + k2 + jnp.uint32(3)
    for r in (17, 29, 16, 24):
        x0 = x0 + x1
        x1 = _rotl(x1, r) ^ x0
    x0 = x0 + k2
    x1 = x1 + ks2 + jnp.uint32(4)
    for r in (13, 15, 26, 6):
        x0 = x0 + x1
        x1 = _rotl(x1, r) ^ x0
    x0 = x0 + ks2
    x1 = x1 + k1 + jnp.uint32(5)
    return x0 ^ x1


def _mix32(x):
    x = x ^ (x >> jnp.uint32(16))
    x = x * jnp.uint32(0x7FEB352D)
    x = x ^ (x >> jnp.uint32(15))
    x = x * jnp.uint32(0x846CA68B)
    x = x ^ (x >> jnp.uint32(16))
    return x


def _hcomb(h, v):
    return _mix32(h ^ (_mix32(v) + jnp.uint32(0x9E3779B9) + (h << jnp.uint32(6)) + (h >> jnp.uint32(2))))


def _draws_kernel(cutoff_ref, key_ref, sp_ref, out_ref):
    b = pl.program_id(0)
    cutoff = cutoff_ref[b]
    k1 = key_ref[0]
    k2 = key_ref[1]
    sp = sp_ref[0]
    pos = (jax.lax.broadcasted_iota(jnp.int32, (_ROWS, _COLS), 0) * _COLS
           + jax.lax.broadcasted_iota(jnp.int32, (_ROWS, _COLS), 1))
    keep = pos <= cutoff
    for d in range(_ND):
        base = (jnp.uint32(d * _B) + b.astype(jnp.uint32)) * jnp.uint32(_V)
        n = base + pos.astype(jnp.uint32)
        bits = _threefry_mix(k1, k2, n)
        fb = (bits >> jnp.uint32(9)) | jnp.uint32(0x3F800000)
        f = jax.lax.bitcast_convert_type(fb, jnp.float32) - jnp.float32(1.0)
        u = jnp.maximum(_TINY, f + _TINY)
        val = sp / (-jnp.log(u))
        val = jnp.where(keep, val, jnp.float32(-1.0))
        m = jnp.max(val)
        idx = jnp.min(jnp.where(val == m, pos, jnp.int32(_V)))
        out_ref[0, d, :] = jnp.broadcast_to(idx.reshape(1), (128,))


def _emit_kernel(cand_ref, tail_ref, ridx_ref, out_ref):
    cand = cand_ref[...].astype(jnp.uint32)
    lane = jax.lax.broadcasted_iota(jnp.int32, (_B, 128), 1)
    seed = jnp.full((_B, 128), _SEED, dtype=jnp.uint32)
    h = _mix32(seed ^ _mix32(ridx_ref[...].astype(jnp.uint32)))
    for t in range(_PRIOR):
        v = jnp.broadcast_to(tail_ref[:, t:t + 1].astype(jnp.uint32), (_B, 128))
        h = _hcomb(h, v)
    base = _hcomb(_mix32(seed), h)
    cs = _hcomb(base, cand)
    u01 = jnp.clip((cs >> jnp.uint32(8)).astype(jnp.float32) * jnp.float32(1.0 / 16777216.0),
                   jnp.float32(1e-7), jnp.float32(1.0 - 1e-7))
    u01 = jnp.where(lane < _ND, u01, jnp.float32(-1.0))
    m = jnp.max(u01, axis=1, keepdims=True)
    first = jnp.min(jnp.where(u01 == m, lane, jnp.int32(128)), axis=1, keepdims=True)
    tok = jnp.sum(jnp.where(lane == first, cand_ref[...], 0), axis=1, keepdims=True)
    j = pl.program_id(0)
    gcol = jax.lax.broadcasted_iota(jnp.int32, (_B, _OUT_BS), 1) + j * _OUT_BS
    out_ref[...] = jnp.where(gcol == tok, jnp.float32(100000.0), jnp.float32(1e-05))


def kernel(input_ids, logits):
    B, V = logits.shape
    probs = jax.nn.softmax(logits, axis=-1)
    iota = jax.lax.broadcasted_iota(jnp.int32, (B, V), 1)
    sneg, order = jax.lax.sort((-probs, iota), dimension=-1, is_stable=True, num_keys=1)
    sp = -sneg
    cum = jnp.cumsum(sp, axis=-1)
    cutoff = jnp.minimum(jnp.sum((cum < _TOPP).astype(jnp.int32), axis=-1), V - 1)

    skey = jax.random.key(1)
    kd, kr = jax.random.split(skey)
    keydata = jax.random.key_data(kd).astype(jnp.uint32)

    pstar = pl.pallas_call(
        _draws_kernel,
        grid_spec=pltpu.PrefetchScalarGridSpec(
            num_scalar_prefetch=2,
            grid=(B,),
            in_specs=[pl.BlockSpec((1, _ROWS, _COLS), lambda b, *_: (b, 0, 0))],
            out_specs=pl.BlockSpec((1, _ND, 128), lambda b, *_: (b, 0, 0)),
        ),
        out_shape=jax.ShapeDtypeStruct((B, _ND, 128), jnp.int32),
    )(cutoff.astype(jnp.int32), keydata, sp.reshape(B, _ROWS, _COLS))
    draws = pstar[:, :, 0]

    cand = jnp.take_along_axis(order, draws, axis=-1)
    r_idx = jax.random.randint(kr, (B,), 0, _KHASH, dtype=jnp.int32)

    cand_p = jnp.pad(cand, ((0, 0), (0, 128 - _ND)))
    tail_p = jnp.pad(input_ids[:, -_PRIOR:], ((0, 0), (0, 128 - _PRIOR)))
    ridx_p = jnp.broadcast_to(r_idx[:, None], (B, 128))

    nblk = (V + _OUT_BS - 1) // _OUT_BS
    out = pl.pallas_call(
        _emit_kernel,
        grid=(nblk,),
        in_specs=[
            pl.BlockSpec((B, 128), lambda j: (0, 0)),
            pl.BlockSpec((B, 128), lambda j: (0, 0)),
            pl.BlockSpec((B, 128), lambda j: (0, 0)),
        ],
        out_specs=pl.BlockSpec((B, _OUT_BS), lambda j: (0, j)),
        out_shape=jax.ShapeDtypeStruct((B, V), jnp.float32),
    )(cand_p, tail_p, ridx_p)
    return out


# speedup vs baseline: 1.4540x; 1.3525x over previous
import jax
import jax.numpy as jnp
import numpy as np
from jax.experimental import pallas as pl
from jax.experimental.pallas import tpu as pltpu

_V = 1000000
_B = 32
_SEED = 42
_PRIOR = 4
_KHASH = 4
_TOPP = 0.9
_ND = 8
_TINY = np.float32(1.1754943508222875e-38)

_ROWS = 8
_COLS = _V // _ROWS

_OUT_BS = 65536


def _rotl(x, r):
    return (x << jnp.uint32(r)) | (x >> jnp.uint32(32 - r))


def _threefry_mix(k1, k2, n):
    ks2 = k1 ^ k2 ^ jnp.uint32(0x1BD11BDA)
    x0 = jnp.zeros_like(n) + k1
    x1 = n + k2
    for r in (13, 15, 26, 6):
        x0 = x0 + x1
        x1 = _rotl(x1, r) ^ x0
    x0 = x0 + k2
    x1 = x1 + ks2 + jnp.uint32(1)
    for r in (17, 29, 16, 24):
        x0 = x0 + x1
        x1 = _rotl(x1, r) ^ x0
    x0 = x0 + ks2
    x1 = x1 + k1 + jnp.uint32(2)
    for r in (13, 15, 26, 6):
        x0 = x0 + x1
        x1 = _rotl(x1, r) ^ x0
    x0 = x0 + k1
    x1 = x1 + k2 + jnp.uint32(3)
    for r in (17, 29, 16, 24):
        x0 = x0 + x1
        x1 = _rotl(x1, r) ^ x0
    x0 = x0 + k2
    x1 = x1 + ks2 + jnp.uint32(4)
    for r in (13, 15, 26, 6):
        x0 = x0 + x1
        x1 = _rotl(x1, r) ^ x0
    x0 = x0 + ks2
    x1 = x1 + k1 + jnp.uint32(5)
    return x0 ^ x1


def _mix32(x):
    x = x ^ (x >> jnp.uint32(16))
    x = x * jnp.uint32(0x7FEB352D)
    x = x ^ (x >> jnp.uint32(15))
    x = x * jnp.uint32(0x846CA68B)
    x = x ^ (x >> jnp.uint32(16))
    return x


def _hcomb(h, v):
    return _mix32(h ^ (_mix32(v) + jnp.uint32(0x9E3779B9) + (h << jnp.uint32(6)) + (h >> jnp.uint32(2))))


def _draws_kernel(cutoff_ref, key_ref, sp_ref, out_ref):
    b = pl.program_id(0)
    cutoff = cutoff_ref[b]
    k1 = key_ref[0]
    k2 = key_ref[1]
    sp = sp_ref[0]
    pos = (jax.lax.broadcasted_iota(jnp.int32, (_ROWS, _COLS), 0) * _COLS
           + jax.lax.broadcasted_iota(jnp.int32, (_ROWS, _COLS), 1))
    keep = pos <= cutoff
    for d in range(_ND):
        base = (jnp.uint32(d * _B) + b.astype(jnp.uint32)) * jnp.uint32(_V)
        n = base + pos.astype(jnp.uint32)
        bits = _threefry_mix(k1, k2, n)
        fb = (bits >> jnp.uint32(9)) | jnp.uint32(0x3F800000)
        f = jax.lax.bitcast_convert_type(fb, jnp.float32) - jnp.float32(1.0)
        u = jnp.maximum(_TINY, f + _TINY)
        val = sp / (-jnp.log(u))
        val = jnp.where(keep, val, jnp.float32(-1.0))
        m = jnp.max(val)
        idx = jnp.min(jnp.where(val == m, pos, jnp.int32(_V)))
        out_ref[0, d, :] = jnp.broadcast_to(idx.reshape(1), (128,))


_KEXT = 12


def _recover_kernel(pstar_ref, vstar_ref, probs_ref, out_ref):
    b = pl.program_id(0)
    pr = probs_ref[0]
    pos = (jax.lax.broadcasted_iota(jnp.int32, (_ROWS, _COLS), 0) * _COLS
           + jax.lax.broadcasted_iota(jnp.int32, (_ROWS, _COLS), 1))
    for d in range(_ND):
        v = vstar_ref[b, d]
        p = pstar_ref[b, d]
        gt = pr > v
        eq = pr == v
        lo = jnp.sum(gt.astype(jnp.int32))
        j = p - lo
        prev = jnp.int32(-1)
        tok = jnp.int32(-1)
        for k in range(_KEXT):
            nxt = jnp.min(jnp.where(eq & (pos > prev), pos, jnp.int32(_V)))
            tok = jnp.where((k == j) | (k == 0), nxt, tok)
            prev = nxt
        out_ref[0, d, :] = jnp.broadcast_to(tok.reshape(1), (128,))


def _emit_kernel(cand_ref, tail_ref, ridx_ref, out_ref):
    cand = cand_ref[...].astype(jnp.uint32)
    lane = jax.lax.broadcasted_iota(jnp.int32, (_B, 128), 1)
    seed = jnp.full((_B, 128), _SEED, dtype=jnp.uint32)
    h = _mix32(seed ^ _mix32(ridx_ref[...].astype(jnp.uint32)))
    for t in range(_PRIOR):
        v = jnp.broadcast_to(tail_ref[:, t:t + 1].astype(jnp.uint32), (_B, 128))
        h = _hcomb(h, v)
    base = _hcomb(_mix32(seed), h)
    cs = _hcomb(base, cand)
    u01 = jnp.clip((cs >> jnp.uint32(8)).astype(jnp.float32) * jnp.float32(1.0 / 16777216.0),
                   jnp.float32(1e-7), jnp.float32(1.0 - 1e-7))
    u01 = jnp.where(lane < _ND, u01, jnp.float32(-1.0))
    m = jnp.max(u01, axis=1, keepdims=True)
    first = jnp.min(jnp.where(u01 == m, lane, jnp.int32(128)), axis=1, keepdims=True)
    tok = jnp.sum(jnp.where(lane == first, cand_ref[...], 0), axis=1, keepdims=True)
    j = pl.program_id(0)
    gcol = jax.lax.broadcasted_iota(jnp.int32, (_B, _OUT_BS), 1) + j * _OUT_BS
    out_ref[...] = jnp.where(gcol == tok, jnp.float32(100000.0), jnp.float32(1e-05))


def kernel(input_ids, logits):
    B, V = logits.shape
    probs = jax.nn.softmax(logits, axis=-1)
    sp = -jax.lax.sort(-probs, dimension=-1, is_stable=False)
    cum = jnp.cumsum(sp, axis=-1)
    cutoff = jnp.minimum(jnp.sum((cum < _TOPP).astype(jnp.int32), axis=-1), V - 1)

    skey = jax.random.key(1)
    kd, kr = jax.random.split(skey)
    keydata = jax.random.key_data(kd).astype(jnp.uint32)

    pstar = pl.pallas_call(
        _draws_kernel,
        grid_spec=pltpu.PrefetchScalarGridSpec(
            num_scalar_prefetch=2,
            grid=(B,),
            in_specs=[pl.BlockSpec((1, _ROWS, _COLS), lambda b, *_: (b, 0, 0))],
            out_specs=pl.BlockSpec((1, _ND, 128), lambda b, *_: (b, 0, 0)),
        ),
        out_shape=jax.ShapeDtypeStruct((B, _ND, 128), jnp.int32),
    )(cutoff.astype(jnp.int32), keydata, sp.reshape(B, _ROWS, _COLS))
    draws = pstar[:, :, 0]
    vstar = jnp.take_along_axis(sp, draws, axis=-1)

    ctok = pl.pallas_call(
        _recover_kernel,
        grid_spec=pltpu.PrefetchScalarGridSpec(
            num_scalar_prefetch=2,
            grid=(B,),
            in_specs=[pl.BlockSpec((1, _ROWS, _COLS), lambda b, *_: (b, 0, 0))],
            out_specs=pl.BlockSpec((1, _ND, 128), lambda b, *_: (b, 0, 0)),
        ),
        out_shape=jax.ShapeDtypeStruct((B, _ND, 128), jnp.int32),
    )(draws, vstar, probs.reshape(B, _ROWS, _COLS))
    cand = ctok[:, :, 0]
    r_idx = jax.random.randint(kr, (B,), 0, _KHASH, dtype=jnp.int32)

    cand_p = jnp.pad(cand, ((0, 0), (0, 128 - _ND)))
    tail_p = jnp.pad(input_ids[:, -_PRIOR:], ((0, 0), (0, 128 - _PRIOR)))
    ridx_p = jnp.broadcast_to(r_idx[:, None], (B, 128))

    nblk = (V + _OUT_BS - 1) // _OUT_BS
    out = pl.pallas_call(
        _emit_kernel,
        grid=(nblk,),
        in_specs=[
            pl.BlockSpec((B, 128), lambda j: (0, 0)),
            pl.BlockSpec((B, 128), lambda j: (0, 0)),
            pl.BlockSpec((B, 128), lambda j: (0, 0)),
        ],
        out_specs=pl.BlockSpec((B, _OUT_BS), lambda j: (0, j)),
        out_shape=jax.ShapeDtypeStruct((B, V), jnp.float32),
    )(cand_p, tail_p, ridx_p)
    return out


# speedup vs baseline: 1.6947x; 1.1656x over previous
import jax
import jax.numpy as jnp
import numpy as np
from jax.experimental import pallas as pl
from jax.experimental.pallas import tpu as pltpu

_V = 1000000
_B = 32
_SEED = 42
_PRIOR = 4
_KHASH = 4
_TOPP = 0.9
_ND = 8
_TINY = np.float32(1.1754943508222875e-38)

_ROWS = 8
_COLS = _V // _ROWS

_OUT_BS = 65536


def _rotl(x, r):
    return (x << jnp.uint32(r)) | (x >> jnp.uint32(32 - r))


def _threefry_mix(k1, k2, n):
    ks2 = k1 ^ k2 ^ jnp.uint32(0x1BD11BDA)
    x0 = jnp.zeros_like(n) + k1
    x1 = n + k2
    for r in (13, 15, 26, 6):
        x0 = x0 + x1
        x1 = _rotl(x1, r) ^ x0
    x0 = x0 + k2
    x1 = x1 + ks2 + jnp.uint32(1)
    for r in (17, 29, 16, 24):
        x0 = x0 + x1
        x1 = _rotl(x1, r) ^ x0
    x0 = x0 + ks2
    x1 = x1 + k1 + jnp.uint32(2)
    for r in (13, 15, 26, 6):
        x0 = x0 + x1
        x1 = _rotl(x1, r) ^ x0
    x0 = x0 + k1
    x1 = x1 + k2 + jnp.uint32(3)
    for r in (17, 29, 16, 24):
        x0 = x0 + x1
        x1 = _rotl(x1, r) ^ x0
    x0 = x0 + k2
    x1 = x1 + ks2 + jnp.uint32(4)
    for r in (13, 15, 26, 6):
        x0 = x0 + x1
        x1 = _rotl(x1, r) ^ x0
    x0 = x0 + ks2
    x1 = x1 + k1 + jnp.uint32(5)
    return x0 ^ x1


def _mix32(x):
    x = x ^ (x >> jnp.uint32(16))
    x = x * jnp.uint32(0x7FEB352D)
    x = x ^ (x >> jnp.uint32(15))
    x = x * jnp.uint32(0x846CA68B)
    x = x ^ (x >> jnp.uint32(16))
    return x


def _hcomb(h, v):
    return _mix32(h ^ (_mix32(v) + jnp.uint32(0x9E3779B9) + (h << jnp.uint32(6)) + (h >> jnp.uint32(2))))


_NCH = 8
_CROWS = 8
_CCOLS = _V // (_NCH * _CROWS)
_CHUNK = _CROWS * _CCOLS


def _draws_kernel(cutoff_ref, key_ref, sp_ref, out_ref):
    b = pl.program_id(0)
    cutoff = cutoff_ref[b]
    k1 = key_ref[0]
    k2 = key_ref[1]
    cpos = (jax.lax.broadcasted_iota(jnp.int32, (_CROWS, _CCOLS), 0) * _CCOLS
            + jax.lax.broadcasted_iota(jnp.int32, (_CROWS, _CCOLS), 1))
    nch = cutoff // _CHUNK + 1

    def body(ch, carry):
        bv, bi = carry
        blk = sp_ref[0, pl.ds(ch, 1), :, :].reshape(_CROWS, _CCOLS)
        pos = cpos + ch * _CHUNK
        keep = pos <= cutoff
        nbv, nbi = [], []
        for d in range(_ND):
            base = (jnp.uint32(d * _B) + b.astype(jnp.uint32)) * jnp.uint32(_V)
            n = base + pos.astype(jnp.uint32)
            bits = _threefry_mix(k1, k2, n)
            fb = (bits >> jnp.uint32(9)) | jnp.uint32(0x3F800000)
            f = jax.lax.bitcast_convert_type(fb, jnp.float32) - jnp.float32(1.0)
            u = jnp.maximum(_TINY, f + _TINY)
            val = blk / (-jnp.log(u))
            val = jnp.where(keep, val, jnp.float32(-1.0))
            m = jnp.max(val)
            idx = jnp.min(jnp.where(val == m, pos, jnp.int32(_V)))
            upd = m > bv[d]
            nbv.append(jnp.where(upd, m, bv[d]))
            nbi.append(jnp.where(upd, idx, bi[d]))
        return tuple(nbv), tuple(nbi)

    init = (tuple(jnp.float32(-2.0) for _ in range(_ND)),
            tuple(jnp.int32(0) for _ in range(_ND)))
    bv, bi = jax.lax.fori_loop(0, nch, body, init)
    for d in range(_ND):
        out_ref[0, d, :] = jnp.broadcast_to(bi[d].reshape(1), (128,))


_KEXT = 12


def _recover_kernel(pstar_ref, vstar_ref, probs_ref, out_ref):
    b = pl.program_id(0)
    pr = probs_ref[0]
    pos = (jax.lax.broadcasted_iota(jnp.int32, (_ROWS, _COLS), 0) * _COLS
           + jax.lax.broadcasted_iota(jnp.int32, (_ROWS, _COLS), 1))
    for d in range(_ND):
        v = vstar_ref[b, d]
        p = pstar_ref[b, d]
        gt = pr > v
        eq = pr == v
        lo = jnp.sum(gt.astype(jnp.int32))
        j = p - lo
        mn = jnp.min(jnp.where(eq, pos, jnp.int32(_V)))
        out_ref[0, d, :] = jnp.broadcast_to(mn.reshape(1), (128,))

        @pl.when(j > 0)
        def _dup_path():
            prev = mn
            tok = mn
            for k in range(1, _KEXT):
                nxt = jnp.min(jnp.where(eq & (pos > prev), pos, jnp.int32(_V)))
                tok = jnp.where(k == j, nxt, tok)
                prev = nxt
            out_ref[0, d, :] = jnp.broadcast_to(tok.reshape(1), (128,))


def _emit_kernel(cand_ref, tail_ref, ridx_ref, out_ref):
    cand = cand_ref[...].astype(jnp.uint32)
    lane = jax.lax.broadcasted_iota(jnp.int32, (_B, 128), 1)
    seed = jnp.full((_B, 128), _SEED, dtype=jnp.uint32)
    h = _mix32(seed ^ _mix32(ridx_ref[...].astype(jnp.uint32)))
    for t in range(_PRIOR):
        v = jnp.broadcast_to(tail_ref[:, t:t + 1].astype(jnp.uint32), (_B, 128))
        h = _hcomb(h, v)
    base = _hcomb(_mix32(seed), h)
    cs = _hcomb(base, cand)
    u01 = jnp.clip((cs >> jnp.uint32(8)).astype(jnp.float32) * jnp.float32(1.0 / 16777216.0),
                   jnp.float32(1e-7), jnp.float32(1.0 - 1e-7))
    u01 = jnp.where(lane < _ND, u01, jnp.float32(-1.0))
    m = jnp.max(u01, axis=1, keepdims=True)
    first = jnp.min(jnp.where(u01 == m, lane, jnp.int32(128)), axis=1, keepdims=True)
    tok = jnp.sum(jnp.where(lane == first, cand_ref[...], 0), axis=1, keepdims=True)
    j = pl.program_id(0)
    gcol = jax.lax.broadcasted_iota(jnp.int32, (_B, _OUT_BS), 1) + j * _OUT_BS
    out_ref[...] = jnp.where(gcol == tok, jnp.float32(100000.0), jnp.float32(1e-05))


def kernel(input_ids, logits):
    B, V = logits.shape
    probs = jax.nn.softmax(logits, axis=-1)
    sp = -jax.lax.sort(-probs, dimension=-1, is_stable=False)
    cum = jnp.cumsum(sp, axis=-1)
    cutoff = jnp.minimum(jnp.sum((cum < _TOPP).astype(jnp.int32), axis=-1), V - 1)

    skey = jax.random.key(1)
    kd, kr = jax.random.split(skey)
    keydata = jax.random.key_data(kd).astype(jnp.uint32)

    pstar = pl.pallas_call(
        _draws_kernel,
        grid_spec=pltpu.PrefetchScalarGridSpec(
            num_scalar_prefetch=2,
            grid=(B,),
            in_specs=[pl.BlockSpec((1, _NCH, _CROWS, _CCOLS),
                                   lambda b, *_: (b, 0, 0, 0))],
            out_specs=pl.BlockSpec((1, _ND, 128), lambda b, *_: (b, 0, 0)),
        ),
        out_shape=jax.ShapeDtypeStruct((B, _ND, 128), jnp.int32),
    )(cutoff.astype(jnp.int32), keydata, sp.reshape(B, _NCH, _CROWS, _CCOLS))
    draws = pstar[:, :, 0]
    vstar = jnp.take_along_axis(sp, draws, axis=-1)

    ctok = pl.pallas_call(
        _recover_kernel,
        grid_spec=pltpu.PrefetchScalarGridSpec(
            num_scalar_prefetch=2,
            grid=(B,),
            in_specs=[pl.BlockSpec((1, _ROWS, _COLS), lambda b, *_: (b, 0, 0))],
            out_specs=pl.BlockSpec((1, _ND, 128), lambda b, *_: (b, 0, 0)),
        ),
        out_shape=jax.ShapeDtypeStruct((B, _ND, 128), jnp.int32),
    )(draws, vstar, probs.reshape(B, _ROWS, _COLS))
    cand = ctok[:, :, 0]
    r_idx = jax.random.randint(kr, (B,), 0, _KHASH, dtype=jnp.int32)

    cand_p = jnp.pad(cand, ((0, 0), (0, 128 - _ND)))
    tail_p = jnp.pad(input_ids[:, -_PRIOR:], ((0, 0), (0, 128 - _PRIOR)))
    ridx_p = jnp.broadcast_to(r_idx[:, None], (B, 128))

    nblk = (V + _OUT_BS - 1) // _OUT_BS
    out = pl.pallas_call(
        _emit_kernel,
        grid=(nblk,),
        in_specs=[
            pl.BlockSpec((B, 128), lambda j: (0, 0)),
            pl.BlockSpec((B, 128), lambda j: (0, 0)),
            pl.BlockSpec((B, 128), lambda j: (0, 0)),
        ],
        out_specs=pl.BlockSpec((B, _OUT_BS), lambda j: (0, j)),
        out_shape=jax.ShapeDtypeStruct((B, V), jnp.float32),
    )(cand_p, tail_p, ridx_p)
    return out


# speedup vs baseline: 1.7225x; 1.0164x over previous
import jax
import jax.numpy as jnp
import numpy as np
from jax.experimental import pallas as pl
from jax.experimental.pallas import tpu as pltpu

_V = 1000000
_B = 32
_SEED = 42
_PRIOR = 4
_KHASH = 4
_TOPP = 0.9
_ND = 8
_TINY = np.float32(1.1754943508222875e-38)

_ROWS = 8
_COLS = _V // _ROWS

_OUT_BS = 65536


def _rotl(x, r):
    return (x << jnp.uint32(r)) | (x >> jnp.uint32(32 - r))


def _threefry_mix(k1, k2, n):
    ks2 = k1 ^ k2 ^ jnp.uint32(0x1BD11BDA)
    x0 = jnp.zeros_like(n) + k1
    x1 = n + k2
    for r in (13, 15, 26, 6):
        x0 = x0 + x1
        x1 = _rotl(x1, r) ^ x0
    x0 = x0 + k2
    x1 = x1 + ks2 + jnp.uint32(1)
    for r in (17, 29, 16, 24):
        x0 = x0 + x1
        x1 = _rotl(x1, r) ^ x0
    x0 = x0 + ks2
    x1 = x1 + k1 + jnp.uint32(2)
    for r in (13, 15, 26, 6):
        x0 = x0 + x1
        x1 = _rotl(x1, r) ^ x0
    x0 = x0 + k1
    x1 = x1 + k2 + jnp.uint32(3)
    for r in (17, 29, 16, 24):
        x0 = x0 + x1
        x1 = _rotl(x1, r) ^ x0
    x0 = x0 + k2
    x1 = x1 + ks2 + jnp.uint32(4)
    for r in (13, 15, 26, 6):
        x0 = x0 + x1
        x1 = _rotl(x1, r) ^ x0
    x0 = x0 + ks2
    x1 = x1 + k1 + jnp.uint32(5)
    return x0 ^ x1


def _mix32(x):
    x = x ^ (x >> jnp.uint32(16))
    x = x * jnp.uint32(0x7FEB352D)
    x = x ^ (x >> jnp.uint32(15))
    x = x * jnp.uint32(0x846CA68B)
    x = x ^ (x >> jnp.uint32(16))
    return x


def _hcomb(h, v):
    return _mix32(h ^ (_mix32(v) + jnp.uint32(0x9E3779B9) + (h << jnp.uint32(6)) + (h >> jnp.uint32(2))))


_NCH = 25
_CROWS = 8
_CCOLS = _V // (_NCH * _CROWS)
_CHUNK = _CROWS * _CCOLS


def _draws_kernel(cutoff_ref, key_ref, sp_ref, out_ref):
    b = pl.program_id(0)
    cutoff = cutoff_ref[b]
    k1 = key_ref[0]
    k2 = key_ref[1]
    cpos = (jax.lax.broadcasted_iota(jnp.int32, (_CROWS, _CCOLS), 0) * _CCOLS
            + jax.lax.broadcasted_iota(jnp.int32, (_CROWS, _CCOLS), 1))
    nch = cutoff // _CHUNK + 1

    def body(ch, carry):
        bv, bi = carry
        blk = sp_ref[0, pl.ds(ch, 1), :, :].reshape(_CROWS, _CCOLS)
        pos = cpos + ch * _CHUNK
        keep = pos <= cutoff
        nbv, nbi = [], []
        for d in range(_ND):
            base = (jnp.uint32(d * _B) + b.astype(jnp.uint32)) * jnp.uint32(_V)
            n = base + pos.astype(jnp.uint32)
            bits = _threefry_mix(k1, k2, n)
            fb = (bits >> jnp.uint32(9)) | jnp.uint32(0x3F800000)
            f = jax.lax.bitcast_convert_type(fb, jnp.float32) - jnp.float32(1.0)
            u = jnp.maximum(_TINY, f + _TINY)
            val = blk / (-jnp.log(u))
            val = jnp.where(keep, val, jnp.float32(-1.0))
            m = jnp.max(val)
            idx = jnp.min(jnp.where(val == m, pos, jnp.int32(_V)))
            upd = m > bv[d]
            nbv.append(jnp.where(upd, m, bv[d]))
            nbi.append(jnp.where(upd, idx, bi[d]))
        return tuple(nbv), tuple(nbi)

    init = (tuple(jnp.float32(-2.0) for _ in range(_ND)),
            tuple(jnp.int32(0) for _ in range(_ND)))
    bv, bi = jax.lax.fori_loop(0, nch, body, init)
    for d in range(_ND):
        out_ref[0, d, :] = jnp.broadcast_to(bi[d].reshape(1), (128,))


_KEXT = 12


def _recover_kernel(pstar_ref, vstar_ref, probs_ref, out_ref):
    b = pl.program_id(0)
    pr = probs_ref[0]
    pos = (jax.lax.broadcasted_iota(jnp.int32, (_ROWS, _COLS), 0) * _COLS
           + jax.lax.broadcasted_iota(jnp.int32, (_ROWS, _COLS), 1))
    for d in range(_ND):
        v = vstar_ref[b, d]
        p = pstar_ref[b, d]
        gt = pr > v
        eq = pr == v
        lo = jnp.sum(gt.astype(jnp.int32))
        j = p - lo
        mn = jnp.min(jnp.where(eq, pos, jnp.int32(_V)))
        out_ref[0, d, :] = jnp.broadcast_to(mn.reshape(1), (128,))

        @pl.when(j > 0)
        def _dup_path():
            prev = mn
            tok = mn
            for k in range(1, _KEXT):
                nxt = jnp.min(jnp.where(eq & (pos > prev), pos, jnp.int32(_V)))
                tok = jnp.where(k == j, nxt, tok)
                prev = nxt
            out_ref[0, d, :] = jnp.broadcast_to(tok.reshape(1), (128,))


def _emit_kernel(cand_ref, tail_ref, ridx_ref, out_ref):
    cand = cand_ref[...].astype(jnp.uint32)
    lane = jax.lax.broadcasted_iota(jnp.int32, (_B, 128), 1)
    seed = jnp.full((_B, 128), _SEED, dtype=jnp.uint32)
    h = _mix32(seed ^ _mix32(ridx_ref[...].astype(jnp.uint32)))
    for t in range(_PRIOR):
        v = jnp.broadcast_to(tail_ref[:, t:t + 1].astype(jnp.uint32), (_B, 128))
        h = _hcomb(h, v)
    base = _hcomb(_mix32(seed), h)
    cs = _hcomb(base, cand)
    u01 = jnp.clip((cs >> jnp.uint32(8)).astype(jnp.float32) * jnp.float32(1.0 / 16777216.0),
                   jnp.float32(1e-7), jnp.float32(1.0 - 1e-7))
    u01 = jnp.where(lane < _ND, u01, jnp.float32(-1.0))
    m = jnp.max(u01, axis=1, keepdims=True)
    first = jnp.min(jnp.where(u01 == m, lane, jnp.int32(128)), axis=1, keepdims=True)
    tok = jnp.sum(jnp.where(lane == first, cand_ref[...], 0), axis=1, keepdims=True)
    j = pl.program_id(0)
    gcol = jax.lax.broadcasted_iota(jnp.int32, (_B, _OUT_BS), 1) + j * _OUT_BS
    out_ref[...] = jnp.where(gcol == tok, jnp.float32(100000.0), jnp.float32(1e-05))


def kernel(input_ids, logits):
    B, V = logits.shape
    probs = jax.nn.softmax(logits, axis=-1)
    sp = -jax.lax.sort(-probs, dimension=-1, is_stable=False)
    cum = jnp.cumsum(sp, axis=-1)
    cutoff = jnp.minimum(jnp.sum((cum < _TOPP).astype(jnp.int32), axis=-1), V - 1)

    skey = jax.random.key(1)
    kd, kr = jax.random.split(skey)
    keydata = jax.random.key_data(kd).astype(jnp.uint32)

    pstar = pl.pallas_call(
        _draws_kernel,
        grid_spec=pltpu.PrefetchScalarGridSpec(
            num_scalar_prefetch=2,
            grid=(B,),
            in_specs=[pl.BlockSpec((1, _NCH, _CROWS, _CCOLS),
                                   lambda b, *_: (b, 0, 0, 0))],
            out_specs=pl.BlockSpec((1, _ND, 128), lambda b, *_: (b, 0, 0)),
        ),
        out_shape=jax.ShapeDtypeStruct((B, _ND, 128), jnp.int32),
    )(cutoff.astype(jnp.int32), keydata, sp.reshape(B, _NCH, _CROWS, _CCOLS))
    draws = pstar[:, :, 0]
    vstar = jnp.take_along_axis(sp, draws, axis=-1)

    ctok = pl.pallas_call(
        _recover_kernel,
        grid_spec=pltpu.PrefetchScalarGridSpec(
            num_scalar_prefetch=2,
            grid=(B,),
            in_specs=[pl.BlockSpec((1, _ROWS, _COLS), lambda b, *_: (b, 0, 0))],
            out_specs=pl.BlockSpec((1, _ND, 128), lambda b, *_: (b, 0, 0)),
        ),
        out_shape=jax.ShapeDtypeStruct((B, _ND, 128), jnp.int32),
    )(draws, vstar, probs.reshape(B, _ROWS, _COLS))
    cand = ctok[:, :, 0]
    r_idx = jax.random.randint(kr, (B,), 0, _KHASH, dtype=jnp.int32)

    cand_p = jnp.pad(cand, ((0, 0), (0, 128 - _ND)))
    tail_p = jnp.pad(input_ids[:, -_PRIOR:], ((0, 0), (0, 128 - _PRIOR)))
    ridx_p = jnp.broadcast_to(r_idx[:, None], (B, 128))

    nblk = (V + _OUT_BS - 1) // _OUT_BS
    out = pl.pallas_call(
        _emit_kernel,
        grid=(nblk,),
        in_specs=[
            pl.BlockSpec((B, 128), lambda j: (0, 0)),
            pl.BlockSpec((B, 128), lambda j: (0, 0)),
            pl.BlockSpec((B, 128), lambda j: (0, 0)),
        ],
        out_specs=pl.BlockSpec((B, _OUT_BS), lambda j: (0, j)),
        out_shape=jax.ShapeDtypeStruct((B, V), jnp.float32),
    )(cand_p, tail_p, ridx_p)
    return out
